# Initial kernel scaffold; baseline (speedup 1.0000x reference)
import jax
import jax.numpy as jnp
from jax.experimental import pallas as pl


def kernel(x, c_2, u_2, ptr, enc_W1, enc_b1, enc_W2, enc_b2, enc_ln_g, enc_ln_b, pe_W, pe_b, cls, eln_g, eln_b, Wq, Wk, Xi, eta_logit, r_W1, r_b1, r_ln1_g, r_ln1_b, r_W2, r_b2, r_ln2_g, r_ln2_b, r_W3, r_b3):
    raise NotImplementedError("write your pallas kernel here")


# speedup vs baseline: 1.0446x; 1.0446x over previous
import jax
import jax.numpy as jnp
from jax import lax
from jax.experimental import pallas as pl
from jax.experimental.pallas import tpu as pltpu

G, NLOC, IN_DIM, D, PE_K, KMEM, NC, DEG, STEPS = 64, 156, 128, 256, 16, 32, 10, 16, 8
N1 = NLOC + 1
NP = 160
EPG = NLOC * DEG
F32 = jnp.float32


def _build_kernel(c2_ref, u2_ref, A_ref):
    g = pl.program_id(0)
    src = c2_ref[0, 0, :].reshape(1, EPG) - g * NLOC
    dst = u2_ref[0, 0, :].reshape(1, EPG) - g * NLOC
    rows = lax.broadcasted_iota(jnp.int32, (NP, EPG), 0)
    S = (rows == (src + 1)).astype(F32)
    Dm = (rows == (dst + 1)).astype(F32)
    counts = lax.dot_general(S, Dm, (((1,), (1,)), ((), ())),
                             preferred_element_type=F32)
    ri = lax.broadcasted_iota(jnp.int32, (NP, NP), 0)
    ci = lax.broadcasted_iota(jnp.int32, (NP, NP), 1)
    node_r = jnp.logical_and(ri >= 1, ri < N1)
    node_c = jnp.logical_and(ci >= 1, ci < N1)
    cls_edges = jnp.logical_or(jnp.logical_and(ri == 0, node_c),
                               jnp.logical_and(ci == 0, node_r))
    A_ref[0] = jnp.where(jnp.logical_or(counts > 0.0, cls_edges), 1.0, 0.0).astype(F32)


def _ln_fwd(x, g, b):
    m = jnp.mean(x, axis=-1, keepdims=True)
    v = jnp.mean((x - m) ** 2, axis=-1, keepdims=True)
    return g * (x - m) / jnp.sqrt(v + 1e-5) + b


def _main_kernel(xg_ref, A_ref, pe_ref,
                 eW1_ref, eb1_ref, eW2_ref, eb2_ref, elng_ref, elnb_ref,
                 peW_ref, peb_ref, cls_ref, eg_ref, ebv_ref,
                 Wq_ref, Wk_ref, Xi_ref, eta_ref,
                 rW1_ref, rb1_ref, rg1_ref, rbn1_ref,
                 rW2_ref, rb2_ref, rg2_ref, rbn2_ref,
                 rW3_ref, rb3_ref,
                 out_ref, t_ref, dg_ref):
    eW1 = eW1_ref[...]
    eb1 = eb1_ref[...]
    eW2 = eW2_ref[...]
    eb2 = eb2_ref[...]
    elng = elng_ref[...]
    elnb = elnb_ref[...]
    peW = peW_ref[...]
    peb = peb_ref[...]
    cls_row = cls_ref[...]
    eg = eg_ref[...]
    ebv = ebv_ref[...]
    Wq = Wq_ref[...]
    Wk = Wk_ref[...]
    Xi = Xi_ref[...]
    rowv = (lax.broadcasted_iota(jnp.int32, (NP, 1), 0) < N1)

    def enc_body(g, carry):
        xb = xg_ref[pl.ds(g, 1), :, :].reshape(NLOC, IN_DIM)
        h = jax.nn.gelu(jnp.dot(xb, eW1, preferred_element_type=F32) + eb1)
        z = jnp.dot(h, eW2, preferred_element_type=F32) + eb2
        z = _ln_fwd(z, elng, elnb)
        peg = pe_ref[pl.ds(g, 1), :, :].reshape(NP, PE_K)
        pep = jnp.dot(peg, peW, preferred_element_type=F32) + peb
        base = jnp.concatenate([cls_row, z, jnp.zeros((NP - N1, D), F32)], axis=0)
        t_ref[pl.ds(g, 1), :, :] = jnp.where(rowv, base + pep, 0.0).reshape(1, NP, D)
        return carry

    lax.fori_loop(0, G, enc_body, 0)

    eta = 0.25 * jax.nn.sigmoid(eta_ref[0, 0])
    beta = jnp.float32(0.0625)

    def step_body(s, carry):
        def dg_body(g, ssq):
            t = t_ref[pl.ds(g, 1), :, :].reshape(NP, D)
            m = jnp.mean(t, axis=-1, keepdims=True)
            var = jnp.mean((t - m) ** 2, axis=-1, keepdims=True)
            std = jnp.sqrt(var + 1e-5)
            xh = (t - m) / std
            gl = eg * xh + ebv
            q = jnp.dot(gl, Wq, preferred_element_type=F32)
            k = jnp.dot(gl, Wk, preferred_element_type=F32)
            a = beta * lax.dot_general(q, k, (((1,), (1,)), ((), ())),
                                       preferred_element_type=F32)
            Ag = A_ref[pl.ds(g, 1), :, :].reshape(NP, NP)
            a = jnp.where(Ag > 0.5, a, -1e30)
            amax = jnp.max(a, axis=-1, keepdims=True)
            ex = jnp.exp(a - amax)
            sm = ex / jnp.sum(ex, axis=-1, keepdims=True)
            sm = jnp.where(rowv, sm, 0.0)
            dq = -jnp.dot(sm, k, preferred_element_type=F32)
            dk = -lax.dot_general(sm, q, (((0,), (0,)), ((), ())),
                                  preferred_element_type=F32)
            pre = lax.dot_general(gl, Xi, (((1,), (1,)), ((), ())),
                                  preferred_element_type=F32)
            h = jnp.maximum(pre, 0.0)
            u = (lax.dot_general(dq, Wq, (((1,), (1,)), ((), ())),
                                 preferred_element_type=F32)
                 + lax.dot_general(dk, Wk, (((1,), (1,)), ((), ())),
                                   preferred_element_type=F32)
                 - jnp.dot(h, Xi, preferred_element_type=F32))
            w = u * eg
            dt = (w - jnp.mean(w, axis=-1, keepdims=True)
                  - xh * jnp.mean(w * xh, axis=-1, keepdims=True)) / std
            dt = jnp.where(rowv, dt, 0.0)
            dg_ref[pl.ds(g, 1), :, :] = dt.reshape(1, NP, D)
            return ssq + jnp.sum(dt * dt)

        ssq = lax.fori_loop(0, G, dg_body, jnp.float32(0.0))
        gn = jnp.sqrt(ssq + 1e-12)
        sc = eta * jnp.minimum(1.0, 1.0 / gn)

        def upd_body(g, carry2):
            tt = (t_ref[pl.ds(g, 1), :, :] - sc * dg_ref[pl.ds(g, 1), :, :]).reshape(NP, D)
            tn = jnp.sqrt(jnp.sum(tt * tt, axis=-1, keepdims=True) + 1e-12)
            t_ref[pl.ds(g, 1), :, :] = (tt * jnp.minimum(1.0, 10.0 / tn)).reshape(1, NP, D)
            return carry2

        lax.fori_loop(0, G, upd_body, 0)
        return carry

    lax.fori_loop(0, STEPS, step_body, 0)

    c = t_ref[:, 0, :]
    h1 = jax.nn.gelu(jnp.dot(c, rW1_ref[...], preferred_element_type=F32) + rb1_ref[...])
    h1 = _ln_fwd(h1, rg1_ref[...], rbn1_ref[...])
    h2 = jax.nn.gelu(jnp.dot(h1, rW2_ref[...], preferred_element_type=F32) + rb2_ref[...])
    h2 = _ln_fwd(h2, rg2_ref[...], rbn2_ref[...])
    out_ref[...] = jnp.dot(h2, rW3_ref[...], preferred_element_type=F32) + rb3_ref[...]


def kernel(x, c_2, u_2, ptr, enc_W1, enc_b1, enc_W2, enc_b2, enc_ln_g,
           enc_ln_b, pe_W, pe_b, cls, eln_g, eln_b, Wq, Wk, Xi, eta_logit,
           r_W1, r_b1, r_ln1_g, r_ln1_b, r_W2, r_b2, r_ln2_g, r_ln2_b,
           r_W3, r_b3):
    c2r = c_2.reshape(G, 1, EPG)
    u2r = u_2.reshape(G, 1, EPG)
    A = pl.pallas_call(
        _build_kernel,
        grid=(G,),
        in_specs=[pl.BlockSpec((1, 1, EPG), lambda g: (g, 0, 0)),
                  pl.BlockSpec((1, 1, EPG), lambda g: (g, 0, 0))],
        out_specs=pl.BlockSpec((1, NP, NP), lambda g: (g, 0, 0)),
        out_shape=jax.ShapeDtypeStruct((G, NP, NP), F32),
    )(c2r, u2r)

    Ad = A[:, :N1, :N1]
    Af = jnp.maximum(Ad, jnp.swapaxes(Ad, 1, 2))
    deg = Af.sum(-1)
    dinv = 1.0 / jnp.sqrt(jnp.maximum(deg, 1.0))
    L = jnp.eye(N1, dtype=F32)[None] - dinv[:, :, None] * Af * dinv[:, None, :]
    _, v = jnp.linalg.eigh(L)
    pe = v[:, :, 1:PE_K + 1]
    pe_pad = jnp.pad(pe, ((0, 0), (0, NP - N1), (0, 0)))

    xg = x.reshape(G, NLOC, IN_DIM)
    row2 = lambda a: a.reshape(1, -1)
    args = (xg, A, pe_pad,
            enc_W1, row2(enc_b1), enc_W2, row2(enc_b2), row2(enc_ln_g), row2(enc_ln_b),
            pe_W, row2(pe_b), cls, row2(eln_g), row2(eln_b),
            Wq, Wk, Xi, eta_logit.reshape(1, 1),
            r_W1, row2(r_b1), row2(r_ln1_g), row2(r_ln1_b),
            r_W2, row2(r_b2), row2(r_ln2_g), row2(r_ln2_b),
            r_W3, row2(r_b3))
    out = pl.pallas_call(
        _main_kernel,
        out_shape=jax.ShapeDtypeStruct((G, NC), F32),
        scratch_shapes=[pltpu.VMEM((G, NP, D), F32),
                        pltpu.VMEM((G, NP, D), F32)],
    )(*args)
    return out


# speedup vs baseline: 1.0458x; 1.0012x over previous
import jax
import jax.numpy as jnp
from jax import lax
from jax.experimental import pallas as pl
from jax.experimental.pallas import tpu as pltpu

G, NLOC, IN_DIM, D, PE_K, KMEM, NC, DEG, STEPS = 64, 156, 128, 256, 16, 32, 10, 16, 8
N1 = NLOC + 1
NP = 160
EPG = NLOC * DEG
F32 = jnp.float32


def _build_kernel(c2_ref, u2_ref, A_ref):
    g = pl.program_id(0)
    src = c2_ref[0, 0, :].reshape(1, EPG) - g * NLOC
    dst = u2_ref[0, 0, :].reshape(1, EPG) - g * NLOC
    rows = lax.broadcasted_iota(jnp.int32, (NP, EPG), 0)
    S = (rows == (src + 1)).astype(F32)
    Dm = (rows == (dst + 1)).astype(F32)
    counts = lax.dot_general(S, Dm, (((1,), (1,)), ((), ())),
                             preferred_element_type=F32)
    ri = lax.broadcasted_iota(jnp.int32, (NP, NP), 0)
    ci = lax.broadcasted_iota(jnp.int32, (NP, NP), 1)
    node_r = jnp.logical_and(ri >= 1, ri < N1)
    node_c = jnp.logical_and(ci >= 1, ci < N1)
    cls_edges = jnp.logical_or(jnp.logical_and(ri == 0, node_c),
                               jnp.logical_and(ci == 0, node_r))
    A_ref[0] = jnp.where(jnp.logical_or(counts > 0.0, cls_edges), 1.0, 0.0).astype(F32)


def _ln_fwd(x, g, b):
    m = jnp.mean(x, axis=-1, keepdims=True)
    v = jnp.mean((x - m) ** 2, axis=-1, keepdims=True)
    return g * (x - m) / jnp.sqrt(v + 1e-5) + b


def _main_kernel(xg_ref, A_ref, pe_ref,
                 eW1_ref, eb1_ref, eW2_ref, eb2_ref, elng_ref, elnb_ref,
                 peW_ref, peb_ref, cls_ref, eg_ref, ebv_ref,
                 Wq_ref, Wk_ref, Xi_ref, eta_ref,
                 rW1_ref, rb1_ref, rg1_ref, rbn1_ref,
                 rW2_ref, rb2_ref, rg2_ref, rbn2_ref,
                 rW3_ref, rb3_ref,
                 out_ref, t_ref, dg_ref):
    eW1 = eW1_ref[...]
    eb1 = eb1_ref[...]
    eW2 = eW2_ref[...]
    eb2 = eb2_ref[...]
    elng = elng_ref[...]
    elnb = elnb_ref[...]
    peW = peW_ref[...]
    peb = peb_ref[...]
    cls_row = cls_ref[...]
    eg = eg_ref[...]
    ebv = ebv_ref[...]
    Wq = Wq_ref[...]
    Wk = Wk_ref[...]
    Xi = Xi_ref[...]
    rowv = (lax.broadcasted_iota(jnp.int32, (NP, 1), 0) < N1)

    def enc_body(g, carry):
        xb = xg_ref[pl.ds(g, 1), :, :].reshape(NLOC, IN_DIM)
        h = jax.nn.gelu(jnp.dot(xb, eW1, preferred_element_type=F32) + eb1)
        z = jnp.dot(h, eW2, preferred_element_type=F32) + eb2
        z = _ln_fwd(z, elng, elnb)
        peg = pe_ref[pl.ds(g, 1), :, :].reshape(NP, PE_K)
        pep = jnp.dot(peg, peW, preferred_element_type=F32) + peb
        base = jnp.concatenate([cls_row, z, jnp.zeros((NP - N1, D), F32)], axis=0)
        t_ref[pl.ds(g, 1), :, :] = jnp.where(rowv, base + pep, 0.0).reshape(1, NP, D)
        return carry

    lax.fori_loop(0, G, enc_body, 0)

    eta = 0.25 * jax.nn.sigmoid(eta_ref[0, 0])
    beta = jnp.float32(0.0625)

    def step_body(s, carry):
        def dg_body(g, ssq):
            t = t_ref[pl.ds(g, 1), :, :].reshape(NP, D)
            m = jnp.mean(t, axis=-1, keepdims=True)
            var = jnp.mean((t - m) ** 2, axis=-1, keepdims=True)
            std = jnp.sqrt(var + 1e-5)
            xh = (t - m) / std
            gl = eg * xh + ebv
            q = jnp.dot(gl, Wq, preferred_element_type=F32)
            k = jnp.dot(gl, Wk, preferred_element_type=F32)
            a = beta * lax.dot_general(q, k, (((1,), (1,)), ((), ())),
                                       preferred_element_type=F32)
            Ag = A_ref[pl.ds(g, 1), :, :].reshape(NP, NP)
            a = jnp.where(Ag > 0.5, a, -1e30)
            amax = jnp.max(a, axis=-1, keepdims=True)
            ex = jnp.exp(a - amax)
            sm = ex / jnp.sum(ex, axis=-1, keepdims=True)
            sm = jnp.where(rowv, sm, 0.0)
            dq = -jnp.dot(sm, k, preferred_element_type=F32)
            dk = -lax.dot_general(sm, q, (((0,), (0,)), ((), ())),
                                  preferred_element_type=F32)
            pre = lax.dot_general(gl, Xi, (((1,), (1,)), ((), ())),
                                  preferred_element_type=F32)
            h = jnp.maximum(pre, 0.0)
            u = (lax.dot_general(dq, Wq, (((1,), (1,)), ((), ())),
                                 preferred_element_type=F32)
                 + lax.dot_general(dk, Wk, (((1,), (1,)), ((), ())),
                                   preferred_element_type=F32)
                 - jnp.dot(h, Xi, preferred_element_type=F32))
            w = u * eg
            dt = (w - jnp.mean(w, axis=-1, keepdims=True)
                  - xh * jnp.mean(w * xh, axis=-1, keepdims=True)) / std
            dt = jnp.where(rowv, dt, 0.0)
            dg_ref[pl.ds(g, 1), :, :] = dt.reshape(1, NP, D)
            return ssq + jnp.sum(dt * dt)

        ssq = lax.fori_loop(0, G, dg_body, jnp.float32(0.0))
        gn = jnp.sqrt(ssq + 1e-12)
        sc = eta * jnp.minimum(1.0, 1.0 / gn)

        def upd_body(g, carry2):
            tt = (t_ref[pl.ds(g, 1), :, :] - sc * dg_ref[pl.ds(g, 1), :, :]).reshape(NP, D)
            tn = jnp.sqrt(jnp.sum(tt * tt, axis=-1, keepdims=True) + 1e-12)
            t_ref[pl.ds(g, 1), :, :] = (tt * jnp.minimum(1.0, 10.0 / tn)).reshape(1, NP, D)
            return carry2

        lax.fori_loop(0, G, upd_body, 0)
        return carry

    lax.fori_loop(0, STEPS, step_body, 0)

    c = t_ref[:, 0, :]
    h1 = jax.nn.gelu(jnp.dot(c, rW1_ref[...], preferred_element_type=F32) + rb1_ref[...])
    h1 = _ln_fwd(h1, rg1_ref[...], rbn1_ref[...])
    h2 = jax.nn.gelu(jnp.dot(h1, rW2_ref[...], preferred_element_type=F32) + rb2_ref[...])
    h2 = _ln_fwd(h2, rg2_ref[...], rbn2_ref[...])
    out_ref[...] = jnp.dot(h2, rW3_ref[...], preferred_element_type=F32) + rb3_ref[...]


def kernel(x, c_2, u_2, ptr, enc_W1, enc_b1, enc_W2, enc_b2, enc_ln_g,
           enc_ln_b, pe_W, pe_b, cls, eln_g, eln_b, Wq, Wk, Xi, eta_logit,
           r_W1, r_b1, r_ln1_g, r_ln1_b, r_W2, r_b2, r_ln2_g, r_ln2_b,
           r_W3, r_b3):
    c2r = c_2.reshape(G, 1, EPG)
    u2r = u_2.reshape(G, 1, EPG)
    A = pl.pallas_call(
        _build_kernel,
        grid=(G,),
        in_specs=[pl.BlockSpec((1, 1, EPG), lambda g: (g, 0, 0)),
                  pl.BlockSpec((1, 1, EPG), lambda g: (g, 0, 0))],
        out_specs=pl.BlockSpec((1, NP, NP), lambda g: (g, 0, 0)),
        out_shape=jax.ShapeDtypeStruct((G, NP, NP), F32),
    )(c2r, u2r)

    Ad = A[:, :N1, :N1]
    Af = jnp.maximum(Ad, jnp.swapaxes(Ad, 1, 2))
    deg = Af.sum(-1)
    dinv = 1.0 / jnp.sqrt(jnp.maximum(deg, 1.0))
    L = jnp.eye(N1, dtype=F32)[None] - dinv[:, :, None] * Af * dinv[:, None, :]
    _, v1 = jnp.linalg.eigh(L[:G // 2])
    _, v2 = jnp.linalg.eigh(L[G // 2:])
    v = jnp.concatenate([v1, v2], axis=0)
    pe = v[:, :, 1:PE_K + 1]
    pe_pad = jnp.pad(pe, ((0, 0), (0, NP - N1), (0, 0)))

    xg = x.reshape(G, NLOC, IN_DIM)
    row2 = lambda a: a.reshape(1, -1)
    args = (xg, A, pe_pad,
            enc_W1, row2(enc_b1), enc_W2, row2(enc_b2), row2(enc_ln_g), row2(enc_ln_b),
            pe_W, row2(pe_b), cls, row2(eln_g), row2(eln_b),
            Wq, Wk, Xi, eta_logit.reshape(1, 1),
            r_W1, row2(r_b1), row2(r_ln1_g), row2(r_ln1_b),
            r_W2, row2(r_b2), row2(r_ln2_g), row2(r_ln2_b),
            r_W3, row2(r_b3))
    out = pl.pallas_call(
        _main_kernel,
        out_shape=jax.ShapeDtypeStruct((G, NC), F32),
        scratch_shapes=[pltpu.VMEM((G, NP, D), F32),
                        pltpu.VMEM((G, NP, D), F32)],
    )(*args)
    return out
